# Initial kernel scaffold; baseline (speedup 1.0000x reference)
import jax
import jax.numpy as jnp
from jax.experimental import pallas as pl


def kernel(x, edge_index, W_self1, W_neigh1, b1, W_self2, W_neigh2, b2, W_self3, W_neigh3, b3, W_self4, W_neigh4, b4):
    raise NotImplementedError("write your pallas kernel here")


# speedup vs baseline: 6.1373x; 6.1373x over previous
import functools

import jax
import jax.numpy as jnp
from jax import lax
from jax.experimental import pallas as pl
from jax.experimental.pallas import tpu as pltpu
from jax.experimental.pallas import tpu_sc as plsc

N_NODES = 10000
N_EDGES = 320000
NC = 2
NS = 16
NW = NC * NS
CHUNK = 80
E_PER_W = N_EDGES // NW
NCH = E_PER_W // CHUNK
NPAD = 10240
ROWS_PER_TILE = NPAD // NS
ZR = 16

_MESH = plsc.VectorSubcoreMesh(
    core_axis_name="c", subcore_axis_name="s", num_cores=NC, num_subcores=NS
)


def _zero_fill(ref, n_rows, n_cols):
    z = jnp.zeros((16,), jnp.float32)

    def body(r, _):
        for j in range(n_cols // 16):
            ref[r, pl.ds(j * 16, 16)] = z
        return 0

    lax.fori_loop(0, n_rows, body, 0)


def _make_agg(d_feats):
    out_type = jax.ShapeDtypeStruct((NC, NPAD, d_feats), jnp.float32)
    scratch = [
        pltpu.VMEM((NCH, CHUNK), jnp.int32),
        pltpu.VMEM((NCH, CHUNK), jnp.int32),
        pltpu.VMEM((CHUNK, d_feats), jnp.float32),
        pltpu.VMEM((ZR, d_feats), jnp.float32),
        pltpu.VMEM_SHARED((NPAD, d_feats), jnp.float32),
        pltpu.SemaphoreType.DMA,
        pltpu.SemaphoreType.DMA,
    ]

    def body(p_hbm, src_hbm, dst_hbm, acc_out, src_v, dst_v, rows_v, zacc,
             acc_sh, gsem, ssem):
        c = lax.axis_index("c")
        s = lax.axis_index("s")
        wid = c * NS + s
        row0 = s * ROWS_PER_TILE

        pltpu.sync_copy(src_hbm.at[wid], src_v)
        pltpu.sync_copy(dst_hbm.at[wid], dst_v)

        _zero_fill(zacc, ZR, d_feats)

        def zcopy(k, _):
            pltpu.sync_copy(zacc, acc_sh.at[pl.ds(row0 + k * ZR, ZR)])
            return 0

        lax.fori_loop(0, ROWS_PER_TILE // ZR, zcopy, 0)

        plsc.subcore_barrier()

        def edge_body(j, _):
            pltpu.async_copy(p_hbm.at[src_v.at[j]], rows_v, gsem).wait()
            pltpu.async_copy(rows_v, acc_sh.at[dst_v.at[j]], ssem,
                             add=True).wait()
            return 0

        lax.fori_loop(0, NCH, edge_body, 0)

        plsc.subcore_barrier()

        pltpu.sync_copy(acc_sh.at[pl.ds(row0, ROWS_PER_TILE)],
                        acc_out.at[c, pl.ds(row0, ROWS_PER_TILE)])

    return pl.kernel(body, out_type=out_type, mesh=_MESH,
                     scratch_types=tuple(scratch))


def _deg_kernel():
    out_type = jax.ShapeDtypeStruct((NC, NPAD, 128), jnp.float32)
    scratch = [
        pltpu.VMEM((NCH, CHUNK), jnp.int32),
        pltpu.VMEM((CHUNK, 128), jnp.float32),
        pltpu.VMEM((ZR, 128), jnp.float32),
        pltpu.VMEM_SHARED((NPAD, 128), jnp.float32),
        pltpu.SemaphoreType.DMA,
    ]

    def body(dst_hbm, deg_out, dst_v, ones_v, zdeg, deg_sh, dsem):
        c = lax.axis_index("c")
        s = lax.axis_index("s")
        wid = c * NS + s
        row0 = s * ROWS_PER_TILE

        pltpu.sync_copy(dst_hbm.at[wid], dst_v)
        _zero_fill(zdeg, ZR, 128)

        one = jnp.ones((16,), jnp.float32)

        def ones_body(r, _):
            for j in range(8):
                ones_v[r, pl.ds(j * 16, 16)] = one
            return 0

        lax.fori_loop(0, CHUNK, ones_body, 0)

        def zcopy(k, _):
            pltpu.sync_copy(zdeg, deg_sh.at[pl.ds(row0 + k * ZR, ZR)])
            return 0

        lax.fori_loop(0, ROWS_PER_TILE // ZR, zcopy, 0)

        plsc.subcore_barrier()

        def edge_body(j, _):
            pltpu.async_copy(ones_v, deg_sh.at[dst_v.at[j]], dsem,
                             add=True).wait()
            return 0

        lax.fori_loop(0, NCH, edge_body, 0)

        plsc.subcore_barrier()

        pltpu.sync_copy(deg_sh.at[pl.ds(row0, ROWS_PER_TILE)],
                        deg_out.at[c, pl.ds(row0, ROWS_PER_TILE)])

    return pl.kernel(body, out_type=out_type, mesh=_MESH,
                     scratch_types=tuple(scratch))


def _proj_kernel(h_ref, wn_ref, ws_ref, b_ref, p_ref, s_ref):
    hb = h_ref[...]
    p_ref[...] = jnp.dot(hb, wn_ref[...], preferred_element_type=jnp.float32)
    s_ref[...] = (jnp.dot(hb, ws_ref[...], preferred_element_type=jnp.float32)
                  + b_ref[...])


def _proj(h, wn_t, ws_t, b2d, d_out, block_rows=400):
    grid = (N_NODES // block_rows,)
    d_in = h.shape[1]
    return pl.pallas_call(
        _proj_kernel,
        grid=grid,
        in_specs=[
            pl.BlockSpec((block_rows, d_in), lambda i: (i, 0)),
            pl.BlockSpec((d_in, d_out), lambda i: (0, 0)),
            pl.BlockSpec((d_in, d_out), lambda i: (0, 0)),
            pl.BlockSpec((1, d_out), lambda i: (0, 0)),
        ],
        out_specs=[
            pl.BlockSpec((block_rows, d_out), lambda i: (i, 0)),
            pl.BlockSpec((block_rows, d_out), lambda i: (i, 0)),
        ],
        out_shape=[
            jax.ShapeDtypeStruct((N_NODES, d_out), jnp.float32),
            jax.ShapeDtypeStruct((N_NODES, d_out), jnp.float32),
        ],
    )(h, wn_t, ws_t, b2d)


def _combine_kernel(relu, s_ref, a0_ref, a1_ref, d0_ref, d1_ref, o_ref):
    deg = d0_ref[:, 0:1] + d1_ref[:, 0:1]
    inv = 1.0 / jnp.maximum(deg, 1.0)
    r = s_ref[...] + (a0_ref[...] + a1_ref[...]) * inv
    o_ref[...] = jnp.maximum(r, 0.0) if relu else r


def _combine(s, acc, deg, relu, block_rows=400):
    d_out = s.shape[1]
    grid = (N_NODES // block_rows,)
    return pl.pallas_call(
        functools.partial(_combine_kernel, relu),
        grid=grid,
        in_specs=[
            pl.BlockSpec((block_rows, d_out), lambda i: (i, 0)),
            pl.BlockSpec((block_rows, d_out), lambda i: (i, 0)),
            pl.BlockSpec((block_rows, d_out), lambda i: (i, 0)),
            pl.BlockSpec((block_rows, 16), lambda i: (i, 0)),
            pl.BlockSpec((block_rows, 16), lambda i: (i, 0)),
        ],
        out_specs=pl.BlockSpec((block_rows, d_out), lambda i: (i, 0)),
        out_shape=jax.ShapeDtypeStruct((N_NODES, d_out), jnp.float32),
    )(s, acc[0], acc[1], deg[0], deg[1])


def kernel(x, edge_index, W_self1, W_neigh1, b1, W_self2, W_neigh2, b2,
           W_self3, W_neigh3, b3, W_self4, W_neigh4, b4):
    src = edge_index[0].astype(jnp.int32).reshape(NW, NCH, CHUNK)
    dst = edge_index[1].astype(jnp.int32).reshape(NW, NCH, CHUNK)

    agg128 = _make_agg(128)
    deg = _deg_kernel()(dst)[:, :N_NODES, :16]

    wn4 = jnp.pad(W_neigh4, ((0, 88), (0, 0)))
    ws4 = jnp.pad(W_self4, ((0, 88), (0, 0)))
    b4p = jnp.pad(b4, (0, 88))

    h = x
    for i, (ws, wn, b, d_out, relu) in enumerate([
        (W_self1, W_neigh1, b1, 128, True),
        (W_self2, W_neigh2, b2, 128, True),
        (W_self3, W_neigh3, b3, 128, True),
        (ws4, wn4, b4p, 128, False),
    ]):
        p, s = _proj(h, wn.T, ws.T, b.reshape(1, -1), d_out)
        acc = agg128(p, src, dst)
        h = _combine(s, acc[:, :N_NODES], deg, relu)

    return h[:, :40]


# speedup vs baseline: 9.0385x; 1.4727x over previous
import functools

import jax
import jax.numpy as jnp
from jax import lax
from jax.experimental import pallas as pl
from jax.experimental.pallas import tpu as pltpu
from jax.experimental.pallas import tpu_sc as plsc

N_NODES = 10000
N_EDGES = 320000
NC = 2
NS = 16
NW = NC * NS
CHUNK = 80
E_PER_W = N_EDGES // NW
NCH = E_PER_W // CHUNK
NPAD = 10240
ROWS_PER_TILE = NPAD // NS
ZR = 16

_MESH = plsc.VectorSubcoreMesh(
    core_axis_name="c", subcore_axis_name="s", num_cores=NC, num_subcores=NS
)


def _zero_fill(ref, n_rows, n_cols):
    z = jnp.zeros((16,), jnp.float32)

    def body(r, _):
        for j in range(n_cols // 16):
            ref[r, pl.ds(j * 16, 16)] = z
        return 0

    lax.fori_loop(0, n_rows, body, 0)


def _make_agg(d_feats):
    out_type = jax.ShapeDtypeStruct((NC, NPAD, d_feats), jnp.float32)
    scratch = [
        pltpu.VMEM((E_PER_W,), jnp.int32),
        pltpu.VMEM((E_PER_W,), jnp.int32),
        pltpu.VMEM((CHUNK, d_feats), jnp.float32),
        pltpu.VMEM((CHUNK, d_feats), jnp.float32),
        pltpu.VMEM((ZR, d_feats), jnp.float32),
        pltpu.VMEM_SHARED((NPAD, d_feats), jnp.float32),
        pltpu.SemaphoreType.DMA,
        pltpu.SemaphoreType.DMA,
    ]

    def body(p_hbm, src_hbm, dst_hbm, acc_out, src_v, dst_v, rows_a, rows_b,
             zacc, acc_sh, gsem, ssem):
        c = lax.axis_index("c")
        s = lax.axis_index("s")
        wid = c * NS + s
        row0 = s * ROWS_PER_TILE

        pltpu.sync_copy(src_hbm.at[pl.ds(wid * E_PER_W, E_PER_W)], src_v)
        pltpu.sync_copy(dst_hbm.at[pl.ds(wid * E_PER_W, E_PER_W)], dst_v)

        _zero_fill(zacc, ZR, d_feats)

        def zcopy(k, _):
            pltpu.sync_copy(zacc, acc_sh.at[pl.ds(row0 + k * ZR, ZR)])
            return 0

        lax.fori_loop(0, ROWS_PER_TILE // ZR, zcopy, 0)

        plsc.subcore_barrier()

        def idx(v, j):
            return v.at[pl.ds(j * CHUNK, CHUNK)]

        def do_chunk(j, cur, other):
            @pl.when(j >= 1)
            def _():
                pltpu.make_async_copy(other, acc_sh.at[idx(dst_v, j)],
                                      ssem).wait()

            @pl.when(j + 1 < NCH)
            def _():
                pltpu.async_copy(p_hbm.at[idx(src_v, j + 1)], other, gsem)

            pltpu.make_async_copy(p_hbm.at[idx(src_v, j)], cur, gsem).wait()
            pltpu.async_copy(cur, acc_sh.at[idx(dst_v, j)], ssem, add=True)

        pltpu.async_copy(p_hbm.at[idx(src_v, 0)], rows_a, gsem)

        def pair_body(t, _):
            do_chunk(2 * t, rows_a, rows_b)
            do_chunk(2 * t + 1, rows_b, rows_a)
            return 0

        lax.fori_loop(0, NCH // 2, pair_body, 0)
        do_chunk(NCH - 1, rows_a, rows_b)
        pltpu.make_async_copy(rows_a, acc_sh.at[idx(dst_v, NCH - 1)],
                              ssem).wait()

        plsc.subcore_barrier()

        pltpu.sync_copy(acc_sh.at[pl.ds(row0, ROWS_PER_TILE)],
                        acc_out.at[c, pl.ds(row0, ROWS_PER_TILE)])

    return pl.kernel(body, out_type=out_type, mesh=_MESH,
                     scratch_types=tuple(scratch))


def _deg_kernel():
    out_type = jax.ShapeDtypeStruct((NC, NPAD, 128), jnp.float32)
    scratch = [
        pltpu.VMEM((E_PER_W,), jnp.int32),
        pltpu.VMEM((CHUNK, 128), jnp.float32),
        pltpu.VMEM((ZR, 128), jnp.float32),
        pltpu.VMEM_SHARED((NPAD, 128), jnp.float32),
        pltpu.SemaphoreType.DMA,
    ]

    def body(dst_hbm, deg_out, dst_v, ones_v, zdeg, deg_sh, dsem):
        c = lax.axis_index("c")
        s = lax.axis_index("s")
        wid = c * NS + s
        row0 = s * ROWS_PER_TILE

        pltpu.sync_copy(dst_hbm.at[pl.ds(wid * E_PER_W, E_PER_W)], dst_v)
        _zero_fill(zdeg, ZR, 128)

        one = jnp.ones((16,), jnp.float32)

        def ones_body(r, _):
            for j in range(8):
                ones_v[r, pl.ds(j * 16, 16)] = one
            return 0

        lax.fori_loop(0, CHUNK, ones_body, 0)

        def zcopy(k, _):
            pltpu.sync_copy(zdeg, deg_sh.at[pl.ds(row0 + k * ZR, ZR)])
            return 0

        lax.fori_loop(0, ROWS_PER_TILE // ZR, zcopy, 0)

        plsc.subcore_barrier()

        def edge_body(j, _):
            pltpu.async_copy(ones_v,
                             deg_sh.at[dst_v.at[pl.ds(j * CHUNK, CHUNK)]],
                             dsem, add=True).wait()
            return 0

        lax.fori_loop(0, NCH, edge_body, 0)

        plsc.subcore_barrier()

        pltpu.sync_copy(deg_sh.at[pl.ds(row0, ROWS_PER_TILE)],
                        deg_out.at[c, pl.ds(row0, ROWS_PER_TILE)])

    return pl.kernel(body, out_type=out_type, mesh=_MESH,
                     scratch_types=tuple(scratch))


def _proj_kernel(h_ref, wn_ref, ws_ref, b_ref, p_ref, s_ref):
    hb = h_ref[...]
    p_ref[...] = jnp.dot(hb, wn_ref[...], preferred_element_type=jnp.float32)
    s_ref[...] = (jnp.dot(hb, ws_ref[...], preferred_element_type=jnp.float32)
                  + b_ref[...])


def _proj(h, wn_t, ws_t, b2d, d_out, block_rows=400):
    grid = (N_NODES // block_rows,)
    d_in = h.shape[1]
    return pl.pallas_call(
        _proj_kernel,
        grid=grid,
        in_specs=[
            pl.BlockSpec((block_rows, d_in), lambda i: (i, 0)),
            pl.BlockSpec((d_in, d_out), lambda i: (0, 0)),
            pl.BlockSpec((d_in, d_out), lambda i: (0, 0)),
            pl.BlockSpec((1, d_out), lambda i: (0, 0)),
        ],
        out_specs=[
            pl.BlockSpec((block_rows, d_out), lambda i: (i, 0)),
            pl.BlockSpec((block_rows, d_out), lambda i: (i, 0)),
        ],
        out_shape=[
            jax.ShapeDtypeStruct((N_NODES, d_out), jnp.float32),
            jax.ShapeDtypeStruct((N_NODES, d_out), jnp.float32),
        ],
    )(h, wn_t, ws_t, b2d)


def _combine_kernel(relu, s_ref, a0_ref, a1_ref, d0_ref, d1_ref, o_ref):
    deg = d0_ref[:, 0:1] + d1_ref[:, 0:1]
    inv = 1.0 / jnp.maximum(deg, 1.0)
    r = s_ref[...] + (a0_ref[...] + a1_ref[...]) * inv
    o_ref[...] = jnp.maximum(r, 0.0) if relu else r


def _combine(s, acc, deg, relu, block_rows=400):
    d_out = s.shape[1]
    grid = (N_NODES // block_rows,)
    return pl.pallas_call(
        functools.partial(_combine_kernel, relu),
        grid=grid,
        in_specs=[
            pl.BlockSpec((block_rows, d_out), lambda i: (i, 0)),
            pl.BlockSpec((block_rows, d_out), lambda i: (i, 0)),
            pl.BlockSpec((block_rows, d_out), lambda i: (i, 0)),
            pl.BlockSpec((block_rows, 16), lambda i: (i, 0)),
            pl.BlockSpec((block_rows, 16), lambda i: (i, 0)),
        ],
        out_specs=pl.BlockSpec((block_rows, d_out), lambda i: (i, 0)),
        out_shape=jax.ShapeDtypeStruct((N_NODES, d_out), jnp.float32),
    )(s, acc[0], acc[1], deg[0], deg[1])


def kernel(x, edge_index, W_self1, W_neigh1, b1, W_self2, W_neigh2, b2,
           W_self3, W_neigh3, b3, W_self4, W_neigh4, b4):
    src = edge_index[0].astype(jnp.int32)
    dst = edge_index[1].astype(jnp.int32)

    agg128 = _make_agg(128)
    deg = _deg_kernel()(dst)[:, :N_NODES, :16]

    wn4 = jnp.pad(W_neigh4, ((0, 88), (0, 0)))
    ws4 = jnp.pad(W_self4, ((0, 88), (0, 0)))
    b4p = jnp.pad(b4, (0, 88))

    h = x
    for i, (ws, wn, b, d_out, relu) in enumerate([
        (W_self1, W_neigh1, b1, 128, True),
        (W_self2, W_neigh2, b2, 128, True),
        (W_self3, W_neigh3, b3, 128, True),
        (ws4, wn4, b4p, 128, False),
    ]):
        p, s = _proj(h, wn.T, ws.T, b.reshape(1, -1), d_out)
        acc = agg128(p, src, dst)
        h = _combine(s, acc[:, :N_NODES], deg, relu)

    return h[:, :40]


# speedup vs baseline: 9.7417x; 1.0778x over previous
import functools

import jax
import jax.numpy as jnp
from jax import lax
from jax.experimental import pallas as pl
from jax.experimental.pallas import tpu as pltpu
from jax.experimental.pallas import tpu_sc as plsc

N_NODES = 10000
N_EDGES = 320000
NC = 2
NS = 16
NW = NC * NS
CHUNK = 80
E_PER_W = N_EDGES // NW
NCH = E_PER_W // CHUNK
NPAD = 10240
ROWS_PER_TILE = NPAD // NS
ZR = 16

_MESH = plsc.VectorSubcoreMesh(
    core_axis_name="c", subcore_axis_name="s", num_cores=NC, num_subcores=NS
)


def _zero_fill(ref, n_rows, n_cols):
    z = jnp.zeros((16,), jnp.float32)

    def body(r, _):
        for j in range(n_cols // 16):
            ref[r, pl.ds(j * 16, 16)] = z
        return 0

    lax.fori_loop(0, n_rows, body, 0)


def _make_agg(d_feats):
    out_type = jax.ShapeDtypeStruct((NC, NPAD, d_feats), jnp.float32)
    scratch = [
        pltpu.VMEM((E_PER_W,), jnp.int32),
        pltpu.VMEM((E_PER_W,), jnp.int32),
        pltpu.VMEM((CHUNK, d_feats), jnp.float32),
        pltpu.VMEM((CHUNK, d_feats), jnp.float32),
        pltpu.VMEM((ZR, d_feats), jnp.float32),
        pltpu.VMEM_SHARED((NPAD, d_feats), jnp.float32),
        pltpu.SemaphoreType.DMA,
        pltpu.SemaphoreType.DMA,
    ]

    def body(p_hbm, src_hbm, dst_hbm, acc_out, src_v, dst_v, rows_a, rows_b,
             zacc, acc_sh, gsem, ssem):
        c = lax.axis_index("c")
        s = lax.axis_index("s")
        wid = c * NS + s
        row0 = s * ROWS_PER_TILE

        pltpu.sync_copy(src_hbm.at[pl.ds(wid * E_PER_W, E_PER_W)], src_v)
        pltpu.sync_copy(dst_hbm.at[pl.ds(wid * E_PER_W, E_PER_W)], dst_v)

        _zero_fill(zacc, ZR, d_feats)

        def zfire(k, _):
            pltpu.async_copy(zacc, acc_sh.at[pl.ds(row0 + k * ZR, ZR)], gsem)
            return 0

        def zdrain(k, _):
            pltpu.make_async_copy(zacc, acc_sh.at[pl.ds(row0, ZR)],
                                  gsem).wait()
            return 0

        lax.fori_loop(0, ROWS_PER_TILE // ZR, zfire, 0)
        lax.fori_loop(0, ROWS_PER_TILE // ZR, zdrain, 0)

        plsc.subcore_barrier()

        def idx(v, j):
            return v.at[pl.ds(j * CHUNK, CHUNK)]

        def do_chunk(j, cur, other):
            @pl.when(j >= 1)
            def _():
                pltpu.make_async_copy(other, acc_sh.at[idx(dst_v, j)],
                                      ssem).wait()

            @pl.when(j + 1 < NCH)
            def _():
                pltpu.async_copy(p_hbm.at[idx(src_v, j + 1)], other, gsem)

            pltpu.make_async_copy(p_hbm.at[idx(src_v, j)], cur, gsem).wait()
            pltpu.async_copy(cur, acc_sh.at[idx(dst_v, j)], ssem, add=True)

        pltpu.async_copy(p_hbm.at[idx(src_v, 0)], rows_a, gsem)

        def pair_body(t, _):
            do_chunk(2 * t, rows_a, rows_b)
            do_chunk(2 * t + 1, rows_b, rows_a)
            return 0

        lax.fori_loop(0, NCH // 2, pair_body, 0)
        do_chunk(NCH - 1, rows_a, rows_b)
        pltpu.make_async_copy(rows_a, acc_sh.at[idx(dst_v, NCH - 1)],
                              ssem).wait()

        plsc.subcore_barrier()

        pltpu.sync_copy(acc_sh.at[pl.ds(row0, ROWS_PER_TILE)],
                        acc_out.at[c, pl.ds(row0, ROWS_PER_TILE)])

    return pl.kernel(body, out_type=out_type, mesh=_MESH,
                     scratch_types=tuple(scratch))


def _deg_kernel():
    out_type = jax.ShapeDtypeStruct((NC, NPAD, 128), jnp.float32)
    scratch = [
        pltpu.VMEM((E_PER_W,), jnp.int32),
        pltpu.VMEM((CHUNK, 128), jnp.float32),
        pltpu.VMEM((ZR, 128), jnp.float32),
        pltpu.VMEM_SHARED((NPAD, 128), jnp.float32),
        pltpu.SemaphoreType.DMA,
    ]

    def body(dst_hbm, deg_out, dst_v, ones_v, zdeg, deg_sh, dsem):
        c = lax.axis_index("c")
        s = lax.axis_index("s")
        wid = c * NS + s
        row0 = s * ROWS_PER_TILE

        pltpu.sync_copy(dst_hbm.at[pl.ds(wid * E_PER_W, E_PER_W)], dst_v)
        _zero_fill(zdeg, ZR, 128)

        one = jnp.ones((16,), jnp.float32)

        def ones_body(r, _):
            for j in range(8):
                ones_v[r, pl.ds(j * 16, 16)] = one
            return 0

        lax.fori_loop(0, CHUNK, ones_body, 0)

        def zfire(k, _):
            pltpu.async_copy(zdeg, deg_sh.at[pl.ds(row0 + k * ZR, ZR)], dsem)
            return 0

        def zdrain(k, _):
            pltpu.make_async_copy(zdeg, deg_sh.at[pl.ds(row0, ZR)],
                                  dsem).wait()
            return 0

        lax.fori_loop(0, ROWS_PER_TILE // ZR, zfire, 0)
        lax.fori_loop(0, ROWS_PER_TILE // ZR, zdrain, 0)

        plsc.subcore_barrier()

        def edge_body(j, _):
            @pl.when(j >= 4)
            def _():
                pltpu.make_async_copy(
                    ones_v, deg_sh.at[dst_v.at[pl.ds(0, CHUNK)]],
                    dsem).wait()

            pltpu.async_copy(ones_v,
                             deg_sh.at[dst_v.at[pl.ds(j * CHUNK, CHUNK)]],
                             dsem, add=True)
            return 0

        lax.fori_loop(0, NCH, edge_body, 0)
        for _ in range(4):
            pltpu.make_async_copy(ones_v,
                                  deg_sh.at[dst_v.at[pl.ds(0, CHUNK)]],
                                  dsem).wait()

        plsc.subcore_barrier()

        pltpu.sync_copy(deg_sh.at[pl.ds(row0, ROWS_PER_TILE)],
                        deg_out.at[c, pl.ds(row0, ROWS_PER_TILE)])

    return pl.kernel(body, out_type=out_type, mesh=_MESH,
                     scratch_types=tuple(scratch))


def _proj_kernel(h_ref, wn_ref, ws_ref, b_ref, p_ref, s_ref):
    hb = h_ref[...]
    p_ref[...] = jnp.dot(hb, wn_ref[...], preferred_element_type=jnp.float32)
    s_ref[...] = (jnp.dot(hb, ws_ref[...], preferred_element_type=jnp.float32)
                  + b_ref[...])


def _proj(h, wn_t, ws_t, b2d, d_out, block_rows=400):
    grid = (N_NODES // block_rows,)
    d_in = h.shape[1]
    return pl.pallas_call(
        _proj_kernel,
        grid=grid,
        in_specs=[
            pl.BlockSpec((block_rows, d_in), lambda i: (i, 0)),
            pl.BlockSpec((d_in, d_out), lambda i: (0, 0)),
            pl.BlockSpec((d_in, d_out), lambda i: (0, 0)),
            pl.BlockSpec((1, d_out), lambda i: (0, 0)),
        ],
        out_specs=[
            pl.BlockSpec((block_rows, d_out), lambda i: (i, 0)),
            pl.BlockSpec((block_rows, d_out), lambda i: (i, 0)),
        ],
        out_shape=[
            jax.ShapeDtypeStruct((N_NODES, d_out), jnp.float32),
            jax.ShapeDtypeStruct((N_NODES, d_out), jnp.float32),
        ],
    )(h, wn_t, ws_t, b2d)


def _fused_kernel(s_ref, a0_ref, a1_ref, d0_ref, d1_ref, wn_ref, ws_ref,
                  b_ref, p_ref, s2_ref):
    deg = d0_ref[:, 0:1] + d1_ref[:, 0:1]
    inv = 1.0 / jnp.maximum(deg, 1.0)
    h = jnp.maximum(s_ref[...] + (a0_ref[...] + a1_ref[...]) * inv, 0.0)
    p_ref[...] = jnp.dot(h, wn_ref[...], preferred_element_type=jnp.float32)
    s2_ref[...] = (jnp.dot(h, ws_ref[...], preferred_element_type=jnp.float32)
                   + b_ref[...])


def _fused(s, acc, deg, wn_t, ws_t, b2d, d_out, block_rows=400):
    grid = (N_NODES // block_rows,)
    d_in = s.shape[1]
    return pl.pallas_call(
        _fused_kernel,
        grid=grid,
        in_specs=[
            pl.BlockSpec((block_rows, d_in), lambda i: (i, 0)),
            pl.BlockSpec((block_rows, d_in), lambda i: (i, 0)),
            pl.BlockSpec((block_rows, d_in), lambda i: (i, 0)),
            pl.BlockSpec((block_rows, 16), lambda i: (i, 0)),
            pl.BlockSpec((block_rows, 16), lambda i: (i, 0)),
            pl.BlockSpec((d_in, d_out), lambda i: (0, 0)),
            pl.BlockSpec((d_in, d_out), lambda i: (0, 0)),
            pl.BlockSpec((1, d_out), lambda i: (0, 0)),
        ],
        out_specs=[
            pl.BlockSpec((block_rows, d_out), lambda i: (i, 0)),
            pl.BlockSpec((block_rows, d_out), lambda i: (i, 0)),
        ],
        out_shape=[
            jax.ShapeDtypeStruct((N_NODES, d_out), jnp.float32),
            jax.ShapeDtypeStruct((N_NODES, d_out), jnp.float32),
        ],
    )(s, acc[0], acc[1], deg[0], deg[1], wn_t, ws_t, b2d)


def _combine_kernel(relu, s_ref, a0_ref, a1_ref, d0_ref, d1_ref, o_ref):
    deg = d0_ref[:, 0:1] + d1_ref[:, 0:1]
    inv = 1.0 / jnp.maximum(deg, 1.0)
    r = s_ref[...] + (a0_ref[...] + a1_ref[...]) * inv
    o_ref[...] = jnp.maximum(r, 0.0) if relu else r


def _combine(s, acc, deg, relu, block_rows=400):
    d_out = s.shape[1]
    grid = (N_NODES // block_rows,)
    return pl.pallas_call(
        functools.partial(_combine_kernel, relu),
        grid=grid,
        in_specs=[
            pl.BlockSpec((block_rows, d_out), lambda i: (i, 0)),
            pl.BlockSpec((block_rows, d_out), lambda i: (i, 0)),
            pl.BlockSpec((block_rows, d_out), lambda i: (i, 0)),
            pl.BlockSpec((block_rows, 16), lambda i: (i, 0)),
            pl.BlockSpec((block_rows, 16), lambda i: (i, 0)),
        ],
        out_specs=pl.BlockSpec((block_rows, d_out), lambda i: (i, 0)),
        out_shape=jax.ShapeDtypeStruct((N_NODES, d_out), jnp.float32),
    )(s, acc[0], acc[1], deg[0], deg[1])


def kernel(x, edge_index, W_self1, W_neigh1, b1, W_self2, W_neigh2, b2,
           W_self3, W_neigh3, b3, W_self4, W_neigh4, b4):
    src = edge_index[0].astype(jnp.int32)
    dst = edge_index[1].astype(jnp.int32)

    agg128 = _make_agg(128)
    deg = _deg_kernel()(dst)[:, :N_NODES, :16]

    wn4 = jnp.pad(W_neigh4, ((0, 88), (0, 0)))
    ws4 = jnp.pad(W_self4, ((0, 88), (0, 0)))
    b4p = jnp.pad(b4, (0, 88))

    p, s = _proj(x, W_neigh1.T, W_self1.T, b1.reshape(1, -1), 128)
    acc = agg128(p, src, dst)[:, :N_NODES]
    for wn, ws, b in ((W_neigh2, W_self2, b2), (W_neigh3, W_self3, b3),
                      (wn4, ws4, b4p)):
        p, s = _fused(s, acc, deg, wn.T, ws.T, b.reshape(1, -1), 128)
        acc = agg128(p, src, dst)[:, :N_NODES]
    out = _combine(s, acc, deg, relu=False)
    return out[:, :40]


# speedup vs baseline: 9.7634x; 1.0022x over previous
import functools

import jax
import jax.numpy as jnp
from jax import lax
from jax.experimental import pallas as pl
from jax.experimental.pallas import tpu as pltpu
from jax.experimental.pallas import tpu_sc as plsc

N_NODES = 10000
N_EDGES = 320000
NC = 2
NS = 16
NW = NC * NS
CHUNK = 80
E_PER_W = N_EDGES // NW
NCH = E_PER_W // CHUNK
NPAD = 10240
ROWS_PER_TILE = NPAD // NS
ZR = 16

_MESH = plsc.VectorSubcoreMesh(
    core_axis_name="c", subcore_axis_name="s", num_cores=NC, num_subcores=NS
)


def _zero_fill(ref, n_rows, n_cols):
    z = jnp.zeros((16,), jnp.float32)

    def body(r, _):
        for j in range(n_cols // 16):
            ref[r, pl.ds(j * 16, 16)] = z
        return 0

    lax.fori_loop(0, n_rows, body, 0)


def _make_agg(d_feats):
    out_type = jax.ShapeDtypeStruct((NC, NPAD, d_feats), jnp.float32)
    scratch = [
        pltpu.VMEM((E_PER_W,), jnp.int32),
        pltpu.VMEM((E_PER_W,), jnp.int32),
        pltpu.VMEM((CHUNK, d_feats), jnp.float32),
        pltpu.VMEM((CHUNK, d_feats), jnp.float32),
        pltpu.VMEM((ZR, d_feats), jnp.float32),
        pltpu.VMEM_SHARED((NPAD, d_feats), jnp.float32),
        pltpu.SemaphoreType.DMA,
        pltpu.SemaphoreType.DMA,
    ]

    def body(p_hbm, src_hbm, dst_hbm, acc_out, src_v, dst_v, rows_a, rows_b,
             zacc, acc_sh, gsem, ssem):
        c = lax.axis_index("c")
        s = lax.axis_index("s")
        wid = c * NS + s
        row0 = s * ROWS_PER_TILE

        pltpu.sync_copy(src_hbm.at[pl.ds(wid * E_PER_W, E_PER_W)], src_v)
        pltpu.sync_copy(dst_hbm.at[pl.ds(wid * E_PER_W, E_PER_W)], dst_v)

        _zero_fill(zacc, ZR, d_feats)

        def zfire(k, _):
            pltpu.async_copy(zacc, acc_sh.at[pl.ds(row0 + k * ZR, ZR)], gsem)
            return 0

        def zdrain(k, _):
            pltpu.make_async_copy(zacc, acc_sh.at[pl.ds(row0, ZR)],
                                  gsem).wait()
            return 0

        lax.fori_loop(0, ROWS_PER_TILE // ZR, zfire, 0)
        lax.fori_loop(0, ROWS_PER_TILE // ZR, zdrain, 0)

        plsc.subcore_barrier()

        def idx(v, j):
            return v.at[pl.ds(j * CHUNK, CHUNK)]

        def do_chunk(j, cur, other):
            @pl.when(j >= 1)
            def _():
                pltpu.make_async_copy(other, acc_sh.at[idx(dst_v, j)],
                                      ssem).wait()

            @pl.when(j + 1 < NCH)
            def _():
                pltpu.async_copy(p_hbm.at[idx(src_v, j + 1)], other, gsem)

            pltpu.make_async_copy(p_hbm.at[idx(src_v, j)], cur, gsem).wait()
            pltpu.async_copy(cur, acc_sh.at[idx(dst_v, j)], ssem, add=True,
                             priority=1)

        pltpu.async_copy(p_hbm.at[idx(src_v, 0)], rows_a, gsem)

        def pair_body(t, _):
            do_chunk(2 * t, rows_a, rows_b)
            do_chunk(2 * t + 1, rows_b, rows_a)
            return 0

        lax.fori_loop(0, NCH // 2, pair_body, 0)
        do_chunk(NCH - 1, rows_a, rows_b)
        pltpu.make_async_copy(rows_a, acc_sh.at[idx(dst_v, NCH - 1)],
                              ssem).wait()

        plsc.subcore_barrier()

        pltpu.sync_copy(acc_sh.at[pl.ds(row0, ROWS_PER_TILE)],
                        acc_out.at[c, pl.ds(row0, ROWS_PER_TILE)])

    return pl.kernel(body, out_type=out_type, mesh=_MESH,
                     scratch_types=tuple(scratch))


def _deg_kernel():
    out_type = jax.ShapeDtypeStruct((NC, NPAD, 128), jnp.float32)
    scratch = [
        pltpu.VMEM((E_PER_W,), jnp.int32),
        pltpu.VMEM((CHUNK, 128), jnp.float32),
        pltpu.VMEM((ZR, 128), jnp.float32),
        pltpu.VMEM_SHARED((NPAD, 128), jnp.float32),
        pltpu.SemaphoreType.DMA,
    ]

    def body(dst_hbm, deg_out, dst_v, ones_v, zdeg, deg_sh, dsem):
        c = lax.axis_index("c")
        s = lax.axis_index("s")
        wid = c * NS + s
        row0 = s * ROWS_PER_TILE

        pltpu.sync_copy(dst_hbm.at[pl.ds(wid * E_PER_W, E_PER_W)], dst_v)
        _zero_fill(zdeg, ZR, 128)

        one = jnp.ones((16,), jnp.float32)

        def ones_body(r, _):
            for j in range(8):
                ones_v[r, pl.ds(j * 16, 16)] = one
            return 0

        lax.fori_loop(0, CHUNK, ones_body, 0)

        def zfire(k, _):
            pltpu.async_copy(zdeg, deg_sh.at[pl.ds(row0 + k * ZR, ZR)], dsem)
            return 0

        def zdrain(k, _):
            pltpu.make_async_copy(zdeg, deg_sh.at[pl.ds(row0, ZR)],
                                  dsem).wait()
            return 0

        lax.fori_loop(0, ROWS_PER_TILE // ZR, zfire, 0)
        lax.fori_loop(0, ROWS_PER_TILE // ZR, zdrain, 0)

        plsc.subcore_barrier()

        def edge_body(j, _):
            @pl.when(j >= 4)
            def _():
                pltpu.make_async_copy(
                    ones_v, deg_sh.at[dst_v.at[pl.ds(0, CHUNK)]],
                    dsem).wait()

            pltpu.async_copy(ones_v,
                             deg_sh.at[dst_v.at[pl.ds(j * CHUNK, CHUNK)]],
                             dsem, add=True)
            return 0

        lax.fori_loop(0, NCH, edge_body, 0)
        for _ in range(4):
            pltpu.make_async_copy(ones_v,
                                  deg_sh.at[dst_v.at[pl.ds(0, CHUNK)]],
                                  dsem).wait()

        plsc.subcore_barrier()

        pltpu.sync_copy(deg_sh.at[pl.ds(row0, ROWS_PER_TILE)],
                        deg_out.at[c, pl.ds(row0, ROWS_PER_TILE)])

    return pl.kernel(body, out_type=out_type, mesh=_MESH,
                     scratch_types=tuple(scratch))


def _proj_kernel(h_ref, wn_ref, ws_ref, b_ref, p_ref, s_ref):
    hb = h_ref[...]
    p_ref[...] = jnp.dot(hb, wn_ref[...], preferred_element_type=jnp.float32)
    s_ref[...] = (jnp.dot(hb, ws_ref[...], preferred_element_type=jnp.float32)
                  + b_ref[...])


def _proj(h, wn_t, ws_t, b2d, d_out, block_rows=400):
    grid = (N_NODES // block_rows,)
    d_in = h.shape[1]
    return pl.pallas_call(
        _proj_kernel,
        grid=grid,
        in_specs=[
            pl.BlockSpec((block_rows, d_in), lambda i: (i, 0)),
            pl.BlockSpec((d_in, d_out), lambda i: (0, 0)),
            pl.BlockSpec((d_in, d_out), lambda i: (0, 0)),
            pl.BlockSpec((1, d_out), lambda i: (0, 0)),
        ],
        out_specs=[
            pl.BlockSpec((block_rows, d_out), lambda i: (i, 0)),
            pl.BlockSpec((block_rows, d_out), lambda i: (i, 0)),
        ],
        out_shape=[
            jax.ShapeDtypeStruct((N_NODES, d_out), jnp.float32),
            jax.ShapeDtypeStruct((N_NODES, d_out), jnp.float32),
        ],
    )(h, wn_t, ws_t, b2d)


def _fused_kernel(s_ref, a0_ref, a1_ref, d0_ref, d1_ref, wn_ref, ws_ref,
                  b_ref, p_ref, s2_ref):
    deg = d0_ref[:, 0:1] + d1_ref[:, 0:1]
    inv = 1.0 / jnp.maximum(deg, 1.0)
    h = jnp.maximum(s_ref[...] + (a0_ref[...] + a1_ref[...]) * inv, 0.0)
    p_ref[...] = jnp.dot(h, wn_ref[...], preferred_element_type=jnp.float32)
    s2_ref[...] = (jnp.dot(h, ws_ref[...], preferred_element_type=jnp.float32)
                   + b_ref[...])


def _fused(s, acc, deg, wn_t, ws_t, b2d, d_out, block_rows=400):
    grid = (N_NODES // block_rows,)
    d_in = s.shape[1]
    return pl.pallas_call(
        _fused_kernel,
        grid=grid,
        in_specs=[
            pl.BlockSpec((block_rows, d_in), lambda i: (i, 0)),
            pl.BlockSpec((block_rows, d_in), lambda i: (i, 0)),
            pl.BlockSpec((block_rows, d_in), lambda i: (i, 0)),
            pl.BlockSpec((block_rows, 16), lambda i: (i, 0)),
            pl.BlockSpec((block_rows, 16), lambda i: (i, 0)),
            pl.BlockSpec((d_in, d_out), lambda i: (0, 0)),
            pl.BlockSpec((d_in, d_out), lambda i: (0, 0)),
            pl.BlockSpec((1, d_out), lambda i: (0, 0)),
        ],
        out_specs=[
            pl.BlockSpec((block_rows, d_out), lambda i: (i, 0)),
            pl.BlockSpec((block_rows, d_out), lambda i: (i, 0)),
        ],
        out_shape=[
            jax.ShapeDtypeStruct((N_NODES, d_out), jnp.float32),
            jax.ShapeDtypeStruct((N_NODES, d_out), jnp.float32),
        ],
    )(s, acc[0], acc[1], deg[0], deg[1], wn_t, ws_t, b2d)


def _combine_kernel(relu, s_ref, a0_ref, a1_ref, d0_ref, d1_ref, o_ref):
    deg = d0_ref[:, 0:1] + d1_ref[:, 0:1]
    inv = 1.0 / jnp.maximum(deg, 1.0)
    r = s_ref[...] + (a0_ref[...] + a1_ref[...]) * inv
    o_ref[...] = jnp.maximum(r, 0.0) if relu else r


def _combine(s, acc, deg, relu, block_rows=400):
    d_out = s.shape[1]
    grid = (N_NODES // block_rows,)
    return pl.pallas_call(
        functools.partial(_combine_kernel, relu),
        grid=grid,
        in_specs=[
            pl.BlockSpec((block_rows, d_out), lambda i: (i, 0)),
            pl.BlockSpec((block_rows, d_out), lambda i: (i, 0)),
            pl.BlockSpec((block_rows, d_out), lambda i: (i, 0)),
            pl.BlockSpec((block_rows, 16), lambda i: (i, 0)),
            pl.BlockSpec((block_rows, 16), lambda i: (i, 0)),
        ],
        out_specs=pl.BlockSpec((block_rows, d_out), lambda i: (i, 0)),
        out_shape=jax.ShapeDtypeStruct((N_NODES, d_out), jnp.float32),
    )(s, acc[0], acc[1], deg[0], deg[1])


def kernel(x, edge_index, W_self1, W_neigh1, b1, W_self2, W_neigh2, b2,
           W_self3, W_neigh3, b3, W_self4, W_neigh4, b4):
    src = edge_index[0].astype(jnp.int32)
    dst = edge_index[1].astype(jnp.int32)

    agg128 = _make_agg(128)
    deg = _deg_kernel()(dst)[:, :N_NODES, :16]

    wn4 = jnp.pad(W_neigh4, ((0, 88), (0, 0)))
    ws4 = jnp.pad(W_self4, ((0, 88), (0, 0)))
    b4p = jnp.pad(b4, (0, 88))

    p, s = _proj(x, W_neigh1.T, W_self1.T, b1.reshape(1, -1), 128)
    acc = agg128(p, src, dst)[:, :N_NODES]
    for wn, ws, b in ((W_neigh2, W_self2, b2), (W_neigh3, W_self3, b3),
                      (wn4, ws4, b4p)):
        p, s = _fused(s, acc, deg, wn.T, ws.T, b.reshape(1, -1), 128)
        acc = agg128(p, src, dst)[:, :N_NODES]
    out = _combine(s, acc, deg, relu=False)
    return out[:, :40]


# speedup vs baseline: 10.6900x; 1.0949x over previous
import functools

import jax
import jax.numpy as jnp
from jax import lax
from jax.experimental import pallas as pl
from jax.experimental.pallas import tpu as pltpu
from jax.experimental.pallas import tpu_sc as plsc

N_NODES = 10000
N_EDGES = 320000
NC = 2
NS = 16
NW = NC * NS
CHUNK = 40
E_PER_W = N_EDGES // NW
NCH = E_PER_W // CHUNK
NPAD = 10240
ROWS_PER_TILE = NPAD // NS
ZR = 16

_MESH = plsc.VectorSubcoreMesh(
    core_axis_name="c", subcore_axis_name="s", num_cores=NC, num_subcores=NS
)


def _zero_fill(ref, n_rows, n_cols):
    z = jnp.zeros((16,), jnp.float32)

    def body(r, _):
        for j in range(n_cols // 16):
            ref[r, pl.ds(j * 16, 16)] = z
        return 0

    lax.fori_loop(0, n_rows, body, 0)


def _make_agg(d_feats):
    out_type = jax.ShapeDtypeStruct((NC, NPAD, d_feats), jnp.float32)
    scratch = [
        pltpu.VMEM((E_PER_W,), jnp.int32),
        pltpu.VMEM((E_PER_W,), jnp.int32),
        pltpu.VMEM((CHUNK, d_feats), jnp.float32),
        pltpu.VMEM((CHUNK, d_feats), jnp.float32),
        pltpu.VMEM((CHUNK, d_feats), jnp.float32),
        pltpu.VMEM((CHUNK, d_feats), jnp.float32),
        pltpu.VMEM((ZR, d_feats), jnp.float32),
        pltpu.VMEM_SHARED((NPAD, d_feats), jnp.float32),
        pltpu.SemaphoreType.DMA,
        pltpu.SemaphoreType.DMA,
    ]

    def body(p_hbm, src_hbm, dst_hbm, acc_out, src_v, dst_v, rows_0, rows_1,
             rows_2, rows_3, zacc, acc_sh, gsem, ssem):
        c = lax.axis_index("c")
        s = lax.axis_index("s")
        wid = c * NS + s
        row0 = s * ROWS_PER_TILE

        pltpu.sync_copy(src_hbm.at[pl.ds(wid * E_PER_W, E_PER_W)], src_v)
        pltpu.sync_copy(dst_hbm.at[pl.ds(wid * E_PER_W, E_PER_W)], dst_v)

        _zero_fill(zacc, ZR, d_feats)

        def zfire(k, _):
            pltpu.async_copy(zacc, acc_sh.at[pl.ds(row0 + k * ZR, ZR)], gsem)
            return 0

        def zdrain(k, _):
            pltpu.make_async_copy(zacc, acc_sh.at[pl.ds(row0, ZR)],
                                  gsem).wait()
            return 0

        lax.fori_loop(0, ROWS_PER_TILE // ZR, zfire, 0)
        lax.fori_loop(0, ROWS_PER_TILE // ZR, zdrain, 0)

        plsc.subcore_barrier()

        def idx(v, j):
            return v.at[pl.ds(j * CHUNK, CHUNK)]

        bufs = (rows_0, rows_1, rows_2, rows_3)

        def do_chunk(j, cur, ahead2):
            @pl.when(j >= 2)
            def _():
                pltpu.make_async_copy(ahead2, acc_sh.at[idx(dst_v, j)],
                                      ssem).wait()

            @pl.when(j + 2 < NCH)
            def _():
                pltpu.async_copy(p_hbm.at[idx(src_v, j + 2)], ahead2, gsem)

            pltpu.make_async_copy(p_hbm.at[idx(src_v, j)], cur, gsem).wait()
            pltpu.async_copy(cur, acc_sh.at[idx(dst_v, j)], ssem, add=True,
                             priority=1)

        pltpu.async_copy(p_hbm.at[idx(src_v, 0)], rows_0, gsem)
        pltpu.async_copy(p_hbm.at[idx(src_v, 1)], rows_1, gsem)

        def quad_body(t, _):
            do_chunk(4 * t, rows_0, rows_2)
            do_chunk(4 * t + 1, rows_1, rows_3)
            do_chunk(4 * t + 2, rows_2, rows_0)
            do_chunk(4 * t + 3, rows_3, rows_1)
            return 0

        lax.fori_loop(0, NCH // 4, quad_body, 0)
        do_chunk(NCH - 2, bufs[(NCH - 2) % 4], bufs[NCH % 4])
        do_chunk(NCH - 1, bufs[(NCH - 1) % 4], bufs[(NCH + 1) % 4])
        pltpu.make_async_copy(rows_0, acc_sh.at[idx(dst_v, 0)], ssem).wait()
        pltpu.make_async_copy(rows_0, acc_sh.at[idx(dst_v, 0)], ssem).wait()

        plsc.subcore_barrier()

        pltpu.sync_copy(acc_sh.at[pl.ds(row0, ROWS_PER_TILE)],
                        acc_out.at[c, pl.ds(row0, ROWS_PER_TILE)])

    return pl.kernel(body, out_type=out_type, mesh=_MESH,
                     scratch_types=tuple(scratch))


def _deg_kernel():
    out_type = jax.ShapeDtypeStruct((NC, NPAD, 128), jnp.float32)
    scratch = [
        pltpu.VMEM((E_PER_W,), jnp.int32),
        pltpu.VMEM((CHUNK, 128), jnp.float32),
        pltpu.VMEM((ZR, 128), jnp.float32),
        pltpu.VMEM_SHARED((NPAD, 128), jnp.float32),
        pltpu.SemaphoreType.DMA,
    ]

    def body(dst_hbm, deg_out, dst_v, ones_v, zdeg, deg_sh, dsem):
        c = lax.axis_index("c")
        s = lax.axis_index("s")
        wid = c * NS + s
        row0 = s * ROWS_PER_TILE

        pltpu.sync_copy(dst_hbm.at[pl.ds(wid * E_PER_W, E_PER_W)], dst_v)
        _zero_fill(zdeg, ZR, 128)

        one = jnp.ones((16,), jnp.float32)

        def ones_body(r, _):
            for j in range(8):
                ones_v[r, pl.ds(j * 16, 16)] = one
            return 0

        lax.fori_loop(0, CHUNK, ones_body, 0)

        def zfire(k, _):
            pltpu.async_copy(zdeg, deg_sh.at[pl.ds(row0 + k * ZR, ZR)], dsem)
            return 0

        def zdrain(k, _):
            pltpu.make_async_copy(zdeg, deg_sh.at[pl.ds(row0, ZR)],
                                  dsem).wait()
            return 0

        lax.fori_loop(0, ROWS_PER_TILE // ZR, zfire, 0)
        lax.fori_loop(0, ROWS_PER_TILE // ZR, zdrain, 0)

        plsc.subcore_barrier()

        def edge_body(j, _):
            @pl.when(j >= 4)
            def _():
                pltpu.make_async_copy(
                    ones_v, deg_sh.at[dst_v.at[pl.ds(0, CHUNK)]],
                    dsem).wait()

            pltpu.async_copy(ones_v,
                             deg_sh.at[dst_v.at[pl.ds(j * CHUNK, CHUNK)]],
                             dsem, add=True)
            return 0

        lax.fori_loop(0, NCH, edge_body, 0)
        for _ in range(4):
            pltpu.make_async_copy(ones_v,
                                  deg_sh.at[dst_v.at[pl.ds(0, CHUNK)]],
                                  dsem).wait()

        plsc.subcore_barrier()

        pltpu.sync_copy(deg_sh.at[pl.ds(row0, ROWS_PER_TILE)],
                        deg_out.at[c, pl.ds(row0, ROWS_PER_TILE)])

    return pl.kernel(body, out_type=out_type, mesh=_MESH,
                     scratch_types=tuple(scratch))


def _proj_kernel(h_ref, wn_ref, ws_ref, b_ref, p_ref, s_ref):
    hb = h_ref[...]
    p_ref[...] = jnp.dot(hb, wn_ref[...], preferred_element_type=jnp.float32)
    s_ref[...] = (jnp.dot(hb, ws_ref[...], preferred_element_type=jnp.float32)
                  + b_ref[...])


def _proj(h, wn_t, ws_t, b2d, d_out, block_rows=400):
    grid = (N_NODES // block_rows,)
    d_in = h.shape[1]
    return pl.pallas_call(
        _proj_kernel,
        grid=grid,
        in_specs=[
            pl.BlockSpec((block_rows, d_in), lambda i: (i, 0)),
            pl.BlockSpec((d_in, d_out), lambda i: (0, 0)),
            pl.BlockSpec((d_in, d_out), lambda i: (0, 0)),
            pl.BlockSpec((1, d_out), lambda i: (0, 0)),
        ],
        out_specs=[
            pl.BlockSpec((block_rows, d_out), lambda i: (i, 0)),
            pl.BlockSpec((block_rows, d_out), lambda i: (i, 0)),
        ],
        out_shape=[
            jax.ShapeDtypeStruct((N_NODES, d_out), jnp.float32),
            jax.ShapeDtypeStruct((N_NODES, d_out), jnp.float32),
        ],
    )(h, wn_t, ws_t, b2d)


def _fused_kernel(s_ref, a0_ref, a1_ref, d0_ref, d1_ref, wn_ref, ws_ref,
                  b_ref, p_ref, s2_ref):
    deg = d0_ref[:, 0:1] + d1_ref[:, 0:1]
    inv = 1.0 / jnp.maximum(deg, 1.0)
    h = jnp.maximum(s_ref[...] + (a0_ref[...] + a1_ref[...]) * inv, 0.0)
    p_ref[...] = jnp.dot(h, wn_ref[...], preferred_element_type=jnp.float32)
    s2_ref[...] = (jnp.dot(h, ws_ref[...], preferred_element_type=jnp.float32)
                   + b_ref[...])


def _fused(s, acc, deg, wn_t, ws_t, b2d, d_out, block_rows=400):
    grid = (N_NODES // block_rows,)
    d_in = s.shape[1]
    return pl.pallas_call(
        _fused_kernel,
        grid=grid,
        in_specs=[
            pl.BlockSpec((block_rows, d_in), lambda i: (i, 0)),
            pl.BlockSpec((block_rows, d_in), lambda i: (i, 0)),
            pl.BlockSpec((block_rows, d_in), lambda i: (i, 0)),
            pl.BlockSpec((block_rows, 16), lambda i: (i, 0)),
            pl.BlockSpec((block_rows, 16), lambda i: (i, 0)),
            pl.BlockSpec((d_in, d_out), lambda i: (0, 0)),
            pl.BlockSpec((d_in, d_out), lambda i: (0, 0)),
            pl.BlockSpec((1, d_out), lambda i: (0, 0)),
        ],
        out_specs=[
            pl.BlockSpec((block_rows, d_out), lambda i: (i, 0)),
            pl.BlockSpec((block_rows, d_out), lambda i: (i, 0)),
        ],
        out_shape=[
            jax.ShapeDtypeStruct((N_NODES, d_out), jnp.float32),
            jax.ShapeDtypeStruct((N_NODES, d_out), jnp.float32),
        ],
    )(s, acc[0], acc[1], deg[0], deg[1], wn_t, ws_t, b2d)


def _combine_kernel(relu, s_ref, a0_ref, a1_ref, d0_ref, d1_ref, o_ref):
    deg = d0_ref[:, 0:1] + d1_ref[:, 0:1]
    inv = 1.0 / jnp.maximum(deg, 1.0)
    r = s_ref[...] + (a0_ref[...] + a1_ref[...]) * inv
    o_ref[...] = jnp.maximum(r, 0.0) if relu else r


def _combine(s, acc, deg, relu, block_rows=400):
    d_out = s.shape[1]
    grid = (N_NODES // block_rows,)
    return pl.pallas_call(
        functools.partial(_combine_kernel, relu),
        grid=grid,
        in_specs=[
            pl.BlockSpec((block_rows, d_out), lambda i: (i, 0)),
            pl.BlockSpec((block_rows, d_out), lambda i: (i, 0)),
            pl.BlockSpec((block_rows, d_out), lambda i: (i, 0)),
            pl.BlockSpec((block_rows, 16), lambda i: (i, 0)),
            pl.BlockSpec((block_rows, 16), lambda i: (i, 0)),
        ],
        out_specs=pl.BlockSpec((block_rows, d_out), lambda i: (i, 0)),
        out_shape=jax.ShapeDtypeStruct((N_NODES, d_out), jnp.float32),
    )(s, acc[0], acc[1], deg[0], deg[1])


def kernel(x, edge_index, W_self1, W_neigh1, b1, W_self2, W_neigh2, b2,
           W_self3, W_neigh3, b3, W_self4, W_neigh4, b4):
    src = edge_index[0].astype(jnp.int32)
    dst = edge_index[1].astype(jnp.int32)

    agg128 = _make_agg(128)
    deg = _deg_kernel()(dst)[:, :N_NODES, :16]

    wn4 = jnp.pad(W_neigh4, ((0, 88), (0, 0)))
    ws4 = jnp.pad(W_self4, ((0, 88), (0, 0)))
    b4p = jnp.pad(b4, (0, 88))

    p, s = _proj(x, W_neigh1.T, W_self1.T, b1.reshape(1, -1), 128)
    acc = agg128(p, src, dst)[:, :N_NODES]
    for wn, ws, b in ((W_neigh2, W_self2, b2), (W_neigh3, W_self3, b3),
                      (wn4, ws4, b4p)):
        p, s = _fused(s, acc, deg, wn.T, ws.T, b.reshape(1, -1), 128)
        acc = agg128(p, src, dst)[:, :N_NODES]
    out = _combine(s, acc, deg, relu=False)
    return out[:, :40]


# speedup vs baseline: 11.3946x; 1.0659x over previous
import functools

import jax
import jax.numpy as jnp
from jax import lax
from jax.experimental import pallas as pl
from jax.experimental.pallas import tpu as pltpu
from jax.experimental.pallas import tpu_sc as plsc

N_NODES = 10000
N_EDGES = 320000
NC = 2
NS = 16
NW = NC * NS
CHUNK = 40
E_PER_W = N_EDGES // NW
NCH = E_PER_W // CHUNK
NPAD = 10240
ROWS_PER_TILE = NPAD // NS
ZR = 16

_MESH = plsc.VectorSubcoreMesh(
    core_axis_name="c", subcore_axis_name="s", num_cores=NC, num_subcores=NS
)


def _zero_fill(ref, n_rows, n_cols):
    z = jnp.zeros((16,), jnp.float32)

    def body(r, _):
        for j in range(n_cols // 16):
            ref[r, pl.ds(j * 16, 16)] = z
        return 0

    lax.fori_loop(0, n_rows, body, 0)


def _make_agg(d_feats):
    out_type = jax.ShapeDtypeStruct((NC, NPAD, d_feats), jnp.float32)
    scratch = [
        pltpu.VMEM((E_PER_W,), jnp.int32),
        pltpu.VMEM((E_PER_W,), jnp.int32),
        pltpu.VMEM((CHUNK, d_feats), jnp.float32),
        pltpu.VMEM((CHUNK, d_feats), jnp.float32),
        pltpu.VMEM((CHUNK, d_feats), jnp.float32),
        pltpu.VMEM((CHUNK, d_feats), jnp.float32),
        pltpu.VMEM((CHUNK, d_feats), jnp.float32),
        pltpu.VMEM((ZR, d_feats), jnp.float32),
        pltpu.VMEM_SHARED((NPAD, d_feats), jnp.float32),
        pltpu.SemaphoreType.DMA,
        pltpu.SemaphoreType.DMA,
    ]

    def body(p_hbm, src_hbm, dst_hbm, acc_out, src_v, dst_v, rows_0, rows_1,
             rows_2, rows_3, rows_4, zacc, acc_sh, gsem, ssem):
        c = lax.axis_index("c")
        s = lax.axis_index("s")
        wid = c * NS + s
        row0 = s * ROWS_PER_TILE

        pltpu.sync_copy(src_hbm.at[pl.ds(wid * E_PER_W, E_PER_W)], src_v)
        pltpu.sync_copy(dst_hbm.at[pl.ds(wid * E_PER_W, E_PER_W)], dst_v)

        _zero_fill(zacc, ZR, d_feats)

        def zfire(k, _):
            pltpu.async_copy(zacc, acc_sh.at[pl.ds(row0 + k * ZR, ZR)], gsem)
            return 0

        def zdrain(k, _):
            pltpu.make_async_copy(zacc, acc_sh.at[pl.ds(row0, ZR)],
                                  gsem).wait()
            return 0

        lax.fori_loop(0, ROWS_PER_TILE // ZR, zfire, 0)
        lax.fori_loop(0, ROWS_PER_TILE // ZR, zdrain, 0)

        plsc.subcore_barrier()

        def idx(v, j):
            return v.at[pl.ds(j * CHUNK, CHUNK)]

        bufs = (rows_0, rows_1, rows_2, rows_3, rows_4)

        def do_chunk(j, cur, ahead3):
            @pl.when(j >= 2)
            def _():
                pltpu.make_async_copy(ahead3, acc_sh.at[idx(dst_v, j)],
                                      ssem).wait()

            @pl.when(j + 3 < NCH)
            def _():
                pltpu.async_copy(p_hbm.at[idx(src_v, j + 3)], ahead3, gsem)

            pltpu.make_async_copy(p_hbm.at[idx(src_v, j)], cur, gsem).wait()
            pltpu.async_copy(cur, acc_sh.at[idx(dst_v, j)], ssem, add=True,
                             priority=1)

        pltpu.async_copy(p_hbm.at[idx(src_v, 0)], rows_0, gsem)
        pltpu.async_copy(p_hbm.at[idx(src_v, 1)], rows_1, gsem)
        pltpu.async_copy(p_hbm.at[idx(src_v, 2)], rows_2, gsem)

        def quint_body(t, _):
            do_chunk(5 * t, rows_0, rows_3)
            do_chunk(5 * t + 1, rows_1, rows_4)
            do_chunk(5 * t + 2, rows_2, rows_0)
            do_chunk(5 * t + 3, rows_3, rows_1)
            do_chunk(5 * t + 4, rows_4, rows_2)
            return 0

        lax.fori_loop(0, NCH // 5, quint_body, 0)
        pltpu.make_async_copy(rows_0, acc_sh.at[idx(dst_v, 0)], ssem).wait()
        pltpu.make_async_copy(rows_0, acc_sh.at[idx(dst_v, 0)], ssem).wait()

        plsc.subcore_barrier()

        pltpu.sync_copy(acc_sh.at[pl.ds(row0, ROWS_PER_TILE)],
                        acc_out.at[c, pl.ds(row0, ROWS_PER_TILE)])

    return pl.kernel(body, out_type=out_type, mesh=_MESH,
                     scratch_types=tuple(scratch))


def _deg_kernel():
    out_type = jax.ShapeDtypeStruct((NC, NPAD, 128), jnp.float32)
    scratch = [
        pltpu.VMEM((E_PER_W,), jnp.int32),
        pltpu.VMEM((CHUNK, 128), jnp.float32),
        pltpu.VMEM((ZR, 128), jnp.float32),
        pltpu.VMEM_SHARED((NPAD, 128), jnp.float32),
        pltpu.SemaphoreType.DMA,
    ]

    def body(dst_hbm, deg_out, dst_v, ones_v, zdeg, deg_sh, dsem):
        c = lax.axis_index("c")
        s = lax.axis_index("s")
        wid = c * NS + s
        row0 = s * ROWS_PER_TILE

        pltpu.sync_copy(dst_hbm.at[pl.ds(wid * E_PER_W, E_PER_W)], dst_v)
        _zero_fill(zdeg, ZR, 128)

        one = jnp.ones((16,), jnp.float32)

        def ones_body(r, _):
            for j in range(8):
                ones_v[r, pl.ds(j * 16, 16)] = one
            return 0

        lax.fori_loop(0, CHUNK, ones_body, 0)

        def zfire(k, _):
            pltpu.async_copy(zdeg, deg_sh.at[pl.ds(row0 + k * ZR, ZR)], dsem)
            return 0

        def zdrain(k, _):
            pltpu.make_async_copy(zdeg, deg_sh.at[pl.ds(row0, ZR)],
                                  dsem).wait()
            return 0

        lax.fori_loop(0, ROWS_PER_TILE // ZR, zfire, 0)
        lax.fori_loop(0, ROWS_PER_TILE // ZR, zdrain, 0)

        plsc.subcore_barrier()

        def edge_body(j, _):
            @pl.when(j >= 4)
            def _():
                pltpu.make_async_copy(
                    ones_v, deg_sh.at[dst_v.at[pl.ds(0, CHUNK)]],
                    dsem).wait()

            pltpu.async_copy(ones_v,
                             deg_sh.at[dst_v.at[pl.ds(j * CHUNK, CHUNK)]],
                             dsem, add=True)
            return 0

        lax.fori_loop(0, NCH, edge_body, 0)
        for _ in range(4):
            pltpu.make_async_copy(ones_v,
                                  deg_sh.at[dst_v.at[pl.ds(0, CHUNK)]],
                                  dsem).wait()

        plsc.subcore_barrier()

        pltpu.sync_copy(deg_sh.at[pl.ds(row0, ROWS_PER_TILE)],
                        deg_out.at[c, pl.ds(row0, ROWS_PER_TILE)])

    return pl.kernel(body, out_type=out_type, mesh=_MESH,
                     scratch_types=tuple(scratch))


def _proj_kernel(h_ref, wn_ref, ws_ref, b_ref, p_ref, s_ref):
    hb = h_ref[...]
    p_ref[...] = jnp.dot(hb, wn_ref[...], preferred_element_type=jnp.float32)
    s_ref[...] = (jnp.dot(hb, ws_ref[...], preferred_element_type=jnp.float32)
                  + b_ref[...])


def _proj(h, wn_t, ws_t, b2d, d_out, block_rows=400):
    grid = (N_NODES // block_rows,)
    d_in = h.shape[1]
    return pl.pallas_call(
        _proj_kernel,
        grid=grid,
        in_specs=[
            pl.BlockSpec((block_rows, d_in), lambda i: (i, 0)),
            pl.BlockSpec((d_in, d_out), lambda i: (0, 0)),
            pl.BlockSpec((d_in, d_out), lambda i: (0, 0)),
            pl.BlockSpec((1, d_out), lambda i: (0, 0)),
        ],
        out_specs=[
            pl.BlockSpec((block_rows, d_out), lambda i: (i, 0)),
            pl.BlockSpec((block_rows, d_out), lambda i: (i, 0)),
        ],
        out_shape=[
            jax.ShapeDtypeStruct((N_NODES, d_out), jnp.float32),
            jax.ShapeDtypeStruct((N_NODES, d_out), jnp.float32),
        ],
    )(h, wn_t, ws_t, b2d)


def _fused_kernel(s_ref, a0_ref, a1_ref, d0_ref, d1_ref, wn_ref, ws_ref,
                  b_ref, p_ref, s2_ref):
    deg = d0_ref[:, 0:1] + d1_ref[:, 0:1]
    inv = 1.0 / jnp.maximum(deg, 1.0)
    h = jnp.maximum(s_ref[...] + (a0_ref[...] + a1_ref[...]) * inv, 0.0)
    p_ref[...] = jnp.dot(h, wn_ref[...], preferred_element_type=jnp.float32)
    s2_ref[...] = (jnp.dot(h, ws_ref[...], preferred_element_type=jnp.float32)
                   + b_ref[...])


def _fused(s, acc, deg, wn_t, ws_t, b2d, d_out, block_rows=400):
    grid = (N_NODES // block_rows,)
    d_in = s.shape[1]
    return pl.pallas_call(
        _fused_kernel,
        grid=grid,
        in_specs=[
            pl.BlockSpec((block_rows, d_in), lambda i: (i, 0)),
            pl.BlockSpec((block_rows, d_in), lambda i: (i, 0)),
            pl.BlockSpec((block_rows, d_in), lambda i: (i, 0)),
            pl.BlockSpec((block_rows, 16), lambda i: (i, 0)),
            pl.BlockSpec((block_rows, 16), lambda i: (i, 0)),
            pl.BlockSpec((d_in, d_out), lambda i: (0, 0)),
            pl.BlockSpec((d_in, d_out), lambda i: (0, 0)),
            pl.BlockSpec((1, d_out), lambda i: (0, 0)),
        ],
        out_specs=[
            pl.BlockSpec((block_rows, d_out), lambda i: (i, 0)),
            pl.BlockSpec((block_rows, d_out), lambda i: (i, 0)),
        ],
        out_shape=[
            jax.ShapeDtypeStruct((N_NODES, d_out), jnp.float32),
            jax.ShapeDtypeStruct((N_NODES, d_out), jnp.float32),
        ],
    )(s, acc[0], acc[1], deg[0], deg[1], wn_t, ws_t, b2d)


def _combine_kernel(relu, s_ref, a0_ref, a1_ref, d0_ref, d1_ref, o_ref):
    deg = d0_ref[:, 0:1] + d1_ref[:, 0:1]
    inv = 1.0 / jnp.maximum(deg, 1.0)
    r = s_ref[...] + (a0_ref[...] + a1_ref[...]) * inv
    o_ref[...] = jnp.maximum(r, 0.0) if relu else r


def _combine(s, acc, deg, relu, block_rows=400):
    d_out = s.shape[1]
    grid = (N_NODES // block_rows,)
    return pl.pallas_call(
        functools.partial(_combine_kernel, relu),
        grid=grid,
        in_specs=[
            pl.BlockSpec((block_rows, d_out), lambda i: (i, 0)),
            pl.BlockSpec((block_rows, d_out), lambda i: (i, 0)),
            pl.BlockSpec((block_rows, d_out), lambda i: (i, 0)),
            pl.BlockSpec((block_rows, 16), lambda i: (i, 0)),
            pl.BlockSpec((block_rows, 16), lambda i: (i, 0)),
        ],
        out_specs=pl.BlockSpec((block_rows, d_out), lambda i: (i, 0)),
        out_shape=jax.ShapeDtypeStruct((N_NODES, d_out), jnp.float32),
    )(s, acc[0], acc[1], deg[0], deg[1])


def kernel(x, edge_index, W_self1, W_neigh1, b1, W_self2, W_neigh2, b2,
           W_self3, W_neigh3, b3, W_self4, W_neigh4, b4):
    src = edge_index[0].astype(jnp.int32)
    dst = edge_index[1].astype(jnp.int32)

    agg128 = _make_agg(128)
    deg = _deg_kernel()(dst)[:, :N_NODES, :16]

    wn4 = jnp.pad(W_neigh4, ((0, 88), (0, 0)))
    ws4 = jnp.pad(W_self4, ((0, 88), (0, 0)))
    b4p = jnp.pad(b4, (0, 88))

    p, s = _proj(x, W_neigh1.T, W_self1.T, b1.reshape(1, -1), 128)
    acc = agg128(p, src, dst)[:, :N_NODES]
    for wn, ws, b in ((W_neigh2, W_self2, b2), (W_neigh3, W_self3, b3),
                      (wn4, ws4, b4p)):
        p, s = _fused(s, acc, deg, wn.T, ws.T, b.reshape(1, -1), 128)
        acc = agg128(p, src, dst)[:, :N_NODES]
    out = _combine(s, acc, deg, relu=False)
    return out[:, :40]
